# baseline (speedup 1.0000x reference)
import functools

import jax
import jax.numpy as jnp
from jax import lax
from jax.experimental import pallas as pl
from jax.experimental.pallas import tpu as pltpu
from jax.experimental.pallas import tpu_sc as plsc

D_APP = 256
D_FEAT = 260
BIG_W = 512
OUT_W = 524
TAIL = 12
EB = 32
L = 16


def _features_and_tables(objects_list):
    n, dtot = objects_list.shape
    rows_blk = 1000

    def body(obj_ref, feat_ref, ta_ref, ts_ref, ct_ref):
        x = obj_ref[...]
        app = x[:, :D_APP]
        w = x[:, D_APP:D_APP + 1]
        h = x[:, D_APP + 1:D_APP + 2]
        denom = jnp.maximum(jnp.concatenate([w, h, w, h], axis=1), 1.0)
        pos = x[:, D_APP + 2:D_APP + 6] / denom
        feat_ref[...] = jnp.concatenate([app, pos], axis=1)
        ta_ref[...] = app
        z4 = jnp.zeros((app.shape[0], 4), jnp.float32)
        ts_ref[...] = jnp.concatenate([z4, app[:, :D_APP - 4]], axis=1)
        ct_ref[...] = jnp.concatenate([app[:, D_APP - 4:], pos], axis=1)

    return pl.pallas_call(
        body,
        grid=(n // rows_blk,),
        in_specs=[pl.BlockSpec((rows_blk, dtot), lambda i: (i, 0))],
        out_specs=[pl.BlockSpec((rows_blk, D_FEAT), lambda i: (i, 0)),
                   pl.BlockSpec((rows_blk, D_APP), lambda i: (i, 0)),
                   pl.BlockSpec((rows_blk, D_APP), lambda i: (i, 0)),
                   pl.BlockSpec((rows_blk, 8), lambda i: (i, 0))],
        out_shape=[jax.ShapeDtypeStruct((n, D_FEAT), jnp.float32),
                   jax.ShapeDtypeStruct((n, D_APP), jnp.float32),
                   jax.ShapeDtypeStruct((n, D_APP), jnp.float32),
                   jax.ShapeDtypeStruct((n, 8), jnp.float32)],
    )(objects_list)


def _sqrt16(x):
    bits = plsc.bitcast(x, jnp.int32)
    r = plsc.bitcast(jnp.int32(0x5F3759DF) - lax.shift_right_logical(bits, 1),
                     jnp.float32)
    hx = 0.5 * x
    for _ in range(3):
        r = r * (1.5 - hx * r * r)
    return x * r


def _atan16(a):
    s = a * a
    p = -0.01172120
    for c in (0.05265332, -0.11643287, 0.19354346, -0.33262347, 0.99997726):
        p = p * s + c
    return a * p


def _relation_call(table_a, table_s, ctab, i1, i2, num_edges):
    info = plsc.get_sparse_core_info()
    nw = info.num_cores * info.num_subcores
    num_chunks = num_edges // EB
    slots = -(-num_chunks // nw)
    slots += slots % 2
    npairs = slots // 2
    n_obj = table_a.shape[0]
    mesh = plsc.VectorSubcoreMesh(core_axis_name="c", subcore_axis_name="s")

    @functools.partial(
        pl.kernel, mesh=mesh,
        out_type=jax.ShapeDtypeStruct((num_edges, OUT_W), jnp.float32),
        scratch_types=[
            pltpu.VMEM((EB,), jnp.int32), pltpu.VMEM((EB,), jnp.int32),
            pltpu.VMEM((EB,), jnp.int32), pltpu.VMEM((EB,), jnp.int32),
            pltpu.VMEM((EB, BIG_W), jnp.float32),
            pltpu.VMEM((EB, BIG_W), jnp.float32),
            pltpu.VMEM((EB, TAIL), jnp.float32),
            pltpu.VMEM((EB, TAIL), jnp.float32),
            pltpu.VMEM((n_obj * 8,), jnp.float32),
            pltpu.SemaphoreType.DMA, pltpu.SemaphoreType.DMA,
            pltpu.SemaphoreType.DMA, pltpu.SemaphoreType.DMA,
            pltpu.SemaphoreType.DMA, pltpu.SemaphoreType.DMA,
        ],
        compiler_params=pltpu.CompilerParams(needs_layout_passes=False),
    )
    def k(ta_hbm, ts_hbm, ct_hbm, i1_hbm, i2_hbm, out_hbm,
          ia1, ia2, ib1, ib2, biga, bigb, taila, tailb, ctab_v,
          semi_a, semi_b, semg_a, semg_b, semo_a, semo_b):
        wid = lax.axis_index("s") * info.num_cores + lax.axis_index("c")
        pltpu.sync_copy(ct_hbm, ctab_v)

        bufs = (
            (ia1, ia2, biga, taila, semi_a, semg_a, semo_a),
            (ib1, ib2, bigb, tailb, semi_b, semg_b, semo_b),
        )

        def chunk_base(i, p):
            kk = lax.rem(wid + (2 * i + p) * nw, num_chunks)
            return kk * EB

        def idx_start(p, base):
            x1, x2, _, _, semi, _, _ = bufs[p]
            pltpu.async_copy(i1_hbm.at[pl.ds(base, EB)], x1, semi)
            pltpu.async_copy(i2_hbm.at[pl.ds(base, EB)], x2, semi)

        def idx_wait(p):
            x1, x2, _, _, semi, _, _ = bufs[p]
            pltpu.make_async_copy(i1_hbm.at[pl.ds(0, EB)], x1, semi).wait()
            pltpu.make_async_copy(i2_hbm.at[pl.ds(0, EB)], x2, semi).wait()

        def gathers_start(p):
            x1, x2, big, _, _, semg, _ = bufs[p]
            pltpu.async_copy(ta_hbm.at[x1], big.at[:, pl.ds(0, D_APP)], semg)
            pltpu.async_copy(ts_hbm.at[x2], big.at[:, pl.ds(D_APP, D_APP)],
                             semg)

        def gathers_wait(p):
            x1, x2, big, _, _, semg, _ = bufs[p]
            pltpu.make_async_copy(ta_hbm.at[x1],
                                  big.at[:, pl.ds(0, D_APP)], semg).wait()
            pltpu.make_async_copy(ts_hbm.at[x2],
                                  big.at[:, pl.ds(D_APP, D_APP)],
                                  semg).wait()

        def out_start(p, base):
            _, _, big, tail, _, _, semo = bufs[p]
            pltpu.async_copy(big,
                             out_hbm.at[pl.ds(base, EB), pl.ds(0, BIG_W)],
                             semo)
            pltpu.async_copy(tail,
                             out_hbm.at[pl.ds(base, EB), pl.ds(BIG_W, TAIL)],
                             semo)

        def out_wait(p):
            _, _, big, tail, _, _, semo = bufs[p]
            pltpu.make_async_copy(
                big, out_hbm.at[pl.ds(0, EB), pl.ds(0, BIG_W)], semo).wait()
            pltpu.make_async_copy(
                tail, out_hbm.at[pl.ds(0, EB), pl.ds(BIG_W, TAIL)],
                semo).wait()

        def compute(p):
            x1r, x2r, big, tail, _, _, _ = bufs[p]
            for g in range(EB // L):
                rids = jnp.arange(L, dtype=jnp.int32) + (g * L)
                obj1 = x1r[pl.ds(g * L, L)] * 8
                obj2 = x2r[pl.ds(g * L, L)] * 8

                def ccol(obj, c):
                    return plsc.load_gather(
                        ctab_v, [obj + jnp.full((L,), c, jnp.int32)])

                def put_big(c, v):
                    plsc.store_scatter(
                        big, [rids, jnp.full((L,), c, jnp.int32)], v)

                def put_tail(c, v):
                    plsc.store_scatter(
                        tail, [rids, jnp.full((L,), c, jnp.int32)], v)

                x1 = ccol(obj1, 4)
                y1 = ccol(obj1, 5)
                w1 = ccol(obj1, 6)
                h1 = ccol(obj1, 7)
                x2 = ccol(obj2, 4)
                y2 = ccol(obj2, 5)
                w2 = ccol(obj2, 6)
                h2 = ccol(obj2, 7)
                put_big(D_APP, x1)
                put_big(D_APP + 1, y1)
                put_big(D_APP + 2, w1)
                put_big(D_APP + 3, h1)
                for c in range(4):
                    put_tail(c, ccol(obj2, c))
                put_tail(4, x2)
                put_tail(5, y2)
                put_tail(6, w2)
                put_tail(7, h2)

                dx = ((x1 + w1 * 0.5) - x2) - w2 * 0.5
                dy = ((y1 + h1 * 0.5) - y2) - h2 * 0.5
                dist = _sqrt16(dx * dx + dy * dy)
                ax = jnp.abs(dx)
                ay = jnp.abs(dy)
                a = jnp.minimum(ax, ay) / jnp.maximum(
                    jnp.maximum(ax, ay), 1e-30)
                th = _atan16(a)
                th = jnp.where(ay > ax, (jnp.pi / 2) - th, th)
                put_tail(8, dist)
                put_tail(9, jnp.sign(dy) * th)
                put_tail(10, jnp.sign(x2 - x1))
                put_tail(11, jnp.sign(y2 - y1))

        idx_start(0, chunk_base(0, 0))
        idx_start(1, chunk_base(0, 1))

        def pair_body(i, carry):
            for p in (0, 1):
                idx_wait(p)

                @pl.when(i > 0)
                def _():
                    out_wait(p)
                gathers_start(p)
            for p in (0, 1):
                gathers_wait(p)
                compute(p)
                out_start(p, chunk_base(i, p))

                @pl.when(i + 1 < npairs)
                def _():
                    idx_start(p, chunk_base(i + 1, p))
            return carry

        lax.fori_loop(0, npairs, pair_body, 0)
        out_wait(0)
        out_wait(1)

    return k(table_a, table_s, ctab, i1, i2)


def kernel(objects_list, batch_index, ind0, ind1, ind2):
    feat, table_a, table_s, ctab = _features_and_tables(objects_list)
    i1 = ind1.astype(jnp.int32)
    i2 = ind2.astype(jnp.int32)
    ctf = ctab.reshape(-1)
    e = i1.shape[0]
    h = e // 2
    rel_a = _relation_call(table_a, table_s, ctf, i1[:h], i2[:h], h)
    rel_b = _relation_call(table_a, table_s, ctf, i1[h:], i2[h:], e - h)
    return feat, jnp.concatenate([rel_a, rel_b], axis=0)


# speedup vs baseline: 1.3277x; 1.3277x over previous
import functools

import jax
import jax.numpy as jnp
from jax import lax
from jax.experimental import pallas as pl
from jax.experimental.pallas import tpu as pltpu
from jax.experimental.pallas import tpu_sc as plsc

D_APP = 256
D_FEAT = 260
BIG_W = 512
OUT_W = 524
TAIL = 12
EB = 32
L = 16


def _features_and_tables(objects_list):
    n, dtot = objects_list.shape
    rows_blk = 1000

    def body(obj_ref, feat_ref, ta_ref, ts_ref, ct_ref):
        x = obj_ref[...]
        app = x[:, :D_APP]
        w = x[:, D_APP:D_APP + 1]
        h = x[:, D_APP + 1:D_APP + 2]
        denom = jnp.maximum(jnp.concatenate([w, h, w, h], axis=1), 1.0)
        pos = x[:, D_APP + 2:D_APP + 6] / denom
        feat_ref[...] = jnp.concatenate([app, pos], axis=1)
        ta_ref[...] = app
        z4 = jnp.zeros((app.shape[0], 4), jnp.float32)
        ts_ref[...] = jnp.concatenate([z4, app[:, :D_APP - 4]], axis=1)
        ct_ref[...] = jnp.concatenate([app[:, D_APP - 4:], pos], axis=1)

    return pl.pallas_call(
        body,
        grid=(n // rows_blk,),
        in_specs=[pl.BlockSpec((rows_blk, dtot), lambda i: (i, 0))],
        out_specs=[pl.BlockSpec((rows_blk, D_FEAT), lambda i: (i, 0)),
                   pl.BlockSpec((rows_blk, D_APP), lambda i: (i, 0)),
                   pl.BlockSpec((rows_blk, D_APP), lambda i: (i, 0)),
                   pl.BlockSpec((rows_blk, 8), lambda i: (i, 0))],
        out_shape=[jax.ShapeDtypeStruct((n, D_FEAT), jnp.float32),
                   jax.ShapeDtypeStruct((n, D_APP), jnp.float32),
                   jax.ShapeDtypeStruct((n, D_APP), jnp.float32),
                   jax.ShapeDtypeStruct((n, 8), jnp.float32)],
    )(objects_list)


def _sqrt16(x):
    bits = plsc.bitcast(x, jnp.int32)
    r = plsc.bitcast(jnp.int32(0x5F3759DF) - lax.shift_right_logical(bits, 1),
                     jnp.float32)
    hx = 0.5 * x
    for _ in range(3):
        r = r * (1.5 - hx * r * r)
    return x * r


def _atan16(a):
    s = a * a
    p = -0.01172120
    for c in (0.05265332, -0.11643287, 0.19354346, -0.33262347, 0.99997726):
        p = p * s + c
    return a * p


def _relation_call(table_a, table_s, ctab, i1, i2, num_edges):
    info = plsc.get_sparse_core_info()
    nw = info.num_cores * info.num_subcores
    num_chunks = num_edges // EB
    slots = -(-num_chunks // nw)
    slots += slots % 2
    npairs = slots // 2
    n_obj = table_a.shape[0]
    mesh = plsc.VectorSubcoreMesh(core_axis_name="c", subcore_axis_name="s")

    @functools.partial(
        pl.kernel, mesh=mesh,
        out_type=jax.ShapeDtypeStruct((num_edges, OUT_W), jnp.float32),
        scratch_types=[
            pltpu.VMEM((EB,), jnp.int32), pltpu.VMEM((EB,), jnp.int32),
            pltpu.VMEM((EB,), jnp.int32), pltpu.VMEM((EB,), jnp.int32),
            pltpu.VMEM((EB, BIG_W), jnp.float32),
            pltpu.VMEM((EB, BIG_W), jnp.float32),
            pltpu.VMEM((EB, TAIL), jnp.float32),
            pltpu.VMEM((EB, TAIL), jnp.float32),
            pltpu.VMEM((n_obj * 8,), jnp.float32),
            pltpu.SemaphoreType.DMA, pltpu.SemaphoreType.DMA,
            pltpu.SemaphoreType.DMA, pltpu.SemaphoreType.DMA,
            pltpu.SemaphoreType.DMA, pltpu.SemaphoreType.DMA,
        ],
        compiler_params=pltpu.CompilerParams(needs_layout_passes=False),
    )
    def k(ta_hbm, ts_hbm, ct_hbm, i1_hbm, i2_hbm, out_hbm,
          ia1, ia2, ib1, ib2, biga, bigb, taila, tailb, ctab_v,
          semi_a, semi_b, semg_a, semg_b, semo_a, semo_b):
        wid = lax.axis_index("s") * info.num_cores + lax.axis_index("c")
        pltpu.sync_copy(ct_hbm, ctab_v)

        bufs = (
            (ia1, ia2, biga, taila, semi_a, semg_a, semo_a),
            (ib1, ib2, bigb, tailb, semi_b, semg_b, semo_b),
        )

        def chunk_base(i, p):
            kk = lax.rem(wid + (2 * i + p) * nw, num_chunks)
            return kk * EB

        def idx_start(p, base):
            x1, x2, _, _, semi, _, _ = bufs[p]
            pltpu.async_copy(i1_hbm.at[pl.ds(base, EB)], x1, semi)
            pltpu.async_copy(i2_hbm.at[pl.ds(base, EB)], x2, semi)

        def idx_wait(p):
            x1, x2, _, _, semi, _, _ = bufs[p]
            pltpu.make_async_copy(i1_hbm.at[pl.ds(0, EB)], x1, semi).wait()
            pltpu.make_async_copy(i2_hbm.at[pl.ds(0, EB)], x2, semi).wait()

        def gathers_start(p):
            x1, x2, big, _, _, semg, _ = bufs[p]
            pltpu.async_copy(ta_hbm.at[x1], big.at[:, pl.ds(0, D_APP)], semg)
            pltpu.async_copy(ts_hbm.at[x2], big.at[:, pl.ds(D_APP, D_APP)],
                             semg)

        def gathers_wait(p):
            x1, x2, big, _, _, semg, _ = bufs[p]
            pltpu.make_async_copy(ta_hbm.at[x1],
                                  big.at[:, pl.ds(0, D_APP)], semg).wait()
            pltpu.make_async_copy(ts_hbm.at[x2],
                                  big.at[:, pl.ds(D_APP, D_APP)],
                                  semg).wait()

        def out_start(p, base):
            _, _, big, tail, _, _, semo = bufs[p]
            pltpu.async_copy(big,
                             out_hbm.at[pl.ds(base, EB), pl.ds(0, BIG_W)],
                             semo)
            pltpu.async_copy(tail,
                             out_hbm.at[pl.ds(base, EB), pl.ds(BIG_W, TAIL)],
                             semo)

        def out_wait(p):
            _, _, big, tail, _, _, semo = bufs[p]
            pltpu.make_async_copy(
                big, out_hbm.at[pl.ds(0, EB), pl.ds(0, BIG_W)], semo).wait()
            pltpu.make_async_copy(
                tail, out_hbm.at[pl.ds(0, EB), pl.ds(BIG_W, TAIL)],
                semo).wait()

        def compute(p):
            x1r, x2r, big, tail, _, _, _ = bufs[p]
            for g in range(EB // L):
                rids = jnp.arange(L, dtype=jnp.int32) + (g * L)
                obj1 = x1r[pl.ds(g * L, L)] * 8
                obj2 = x2r[pl.ds(g * L, L)] * 8

                def ccol(obj, c):
                    return plsc.load_gather(
                        ctab_v, [obj + jnp.full((L,), c, jnp.int32)])

                def put_big(c, v):
                    plsc.store_scatter(
                        big, [rids, jnp.full((L,), c, jnp.int32)], v)

                def put_tail(c, v):
                    plsc.store_scatter(
                        tail, [rids, jnp.full((L,), c, jnp.int32)], v)

                x1 = ccol(obj1, 4)
                y1 = ccol(obj1, 5)
                w1 = ccol(obj1, 6)
                h1 = ccol(obj1, 7)
                x2 = ccol(obj2, 4)
                y2 = ccol(obj2, 5)
                w2 = ccol(obj2, 6)
                h2 = ccol(obj2, 7)
                put_big(D_APP, x1)
                put_big(D_APP + 1, y1)
                put_big(D_APP + 2, w1)
                put_big(D_APP + 3, h1)
                for c in range(4):
                    put_tail(c, ccol(obj2, c))
                put_tail(4, x2)
                put_tail(5, y2)
                put_tail(6, w2)
                put_tail(7, h2)

                dx = ((x1 + w1 * 0.5) - x2) - w2 * 0.5
                dy = ((y1 + h1 * 0.5) - y2) - h2 * 0.5
                dist = _sqrt16(dx * dx + dy * dy)
                ax = jnp.abs(dx)
                ay = jnp.abs(dy)
                a = jnp.minimum(ax, ay) / jnp.maximum(
                    jnp.maximum(ax, ay), 1e-30)
                th = _atan16(a)
                th = jnp.where(ay > ax, (jnp.pi / 2) - th, th)
                put_tail(8, dist)
                put_tail(9, jnp.sign(dy) * th)
                put_tail(10, jnp.sign(x2 - x1))
                put_tail(11, jnp.sign(y2 - y1))

        idx_start(0, chunk_base(0, 0))
        idx_start(1, chunk_base(0, 1))

        def pair_body(i, carry):
            for p in (0, 1):
                idx_wait(p)

                @pl.when(i > 0)
                def _():
                    out_wait(p)
                gathers_start(p)
            for p in (0, 1):
                gathers_wait(p)
                compute(p)
                out_start(p, chunk_base(i, p))

                @pl.when(i + 1 < npairs)
                def _():
                    idx_start(p, chunk_base(i + 1, p))
            return carry

        lax.fori_loop(0, npairs, pair_body, 0)
        out_wait(0)
        out_wait(1)

    return k(table_a, table_s, ctab, i1, i2)


def kernel(objects_list, batch_index, ind0, ind1, ind2):
    feat, table_a, table_s, ctab = _features_and_tables(objects_list)
    i1 = ind1.astype(jnp.int32)
    i2 = ind2.astype(jnp.int32)
    rel = _relation_call(table_a, table_s, ctab.reshape(-1), i1, i2,
                         i1.shape[0])
    return feat, rel
